# baseline (speedup 1.0000x reference)
import functools

import jax
import jax.numpy as jnp
from jax import lax
from jax.experimental import pallas as pl
from jax.experimental.pallas import tpu as pltpu
from jax.experimental.pallas import tpu_sc as plsc

ROW_BLOCK = 512


def _dist_block(z, cbt, c2):
    xc = lax.dot_general(z, cbt, (((1,), (0,)), ((), ())),
                         preferred_element_type=jnp.float32)
    x2 = jnp.sum(z * z, axis=1, keepdims=True)
    return (x2 + c2) - 2.0 * xc


def _tc_idx_body(nblocks, k, z_ref, cbt_ref, cw_ref, idx_ref, nw_ref,
                 deadm_ref, hist_ref, c2_ref):
    i = pl.program_id(0)
    cbt = cbt_ref[...]

    @pl.when(i == 0)
    def _():
        c2_ref[...] = jnp.sum(cbt * cbt, axis=0, keepdims=True)

    dist = _dist_block(z_ref[...], cbt, c2_ref[...])
    rowmin = jnp.min(dist, axis=1, keepdims=True)
    mask = dist == rowmin
    iota = lax.broadcasted_iota(jnp.int32, dist.shape, 1)
    idxv = jnp.min(jnp.where(mask, iota, k), axis=1, keepdims=True)
    idx_ref[...] = idxv

    cnt = jnp.sum(mask.astype(jnp.int32), axis=0, keepdims=True)

    @pl.when(i == 0)
    def _():
        hist_ref[...] = cnt

    @pl.when(i > 0)
    def _():
        hist_ref[...] += cnt

    @pl.when(i == nblocks - 1)
    def _():
        nw = cw_ref[...] + hist_ref[...].astype(jnp.float32)
        nw_ref[...] = nw
        total = jnp.sum(nw)
        deadf = jnp.where(nw < total / (100.0 * k), 1.0, 0.0)
        deadm_ref[...] = jnp.broadcast_to(deadf.reshape(k, 1),
                                          deadm_ref.shape)


def _tc_soft_body(z_ref, cbt_ref, soft_ref, c2_ref):
    i = pl.program_id(0)
    cbt = cbt_ref[...]

    @pl.when(i == 0)
    def _():
        c2_ref[...] = jnp.sum(cbt * cbt, axis=0, keepdims=True)

    dist = _dist_block(z_ref[...], cbt, c2_ref[...])
    rowmin = jnp.min(dist, axis=1, keepdims=True)
    t = -100.0 * dist
    m = -100.0 * rowmin
    e = jnp.exp(t - m)
    soft_ref[...] = e * (1.0 / jnp.sum(e, axis=1, keepdims=True))


def _tc_idx_call(z, cbt, cw, n, d, k):
    nblocks = n // ROW_BLOCK
    return pl.pallas_call(
        functools.partial(_tc_idx_body, nblocks, k),
        grid=(nblocks,),
        in_specs=[
            pl.BlockSpec((ROW_BLOCK, d), lambda i: (i, 0)),
            pl.BlockSpec((d, k), lambda i: (0, 0)),
            pl.BlockSpec((1, k), lambda i: (0, 0)),
        ],
        out_specs=[
            pl.BlockSpec((ROW_BLOCK, 1), lambda i: (i, 0)),
            pl.BlockSpec((1, k), lambda i: (0, 0)),
            pl.BlockSpec((k, _L), lambda i: (0, 0)),
        ],
        out_shape=[
            jax.ShapeDtypeStruct((n, 1), jnp.int32),
            jax.ShapeDtypeStruct((1, k), jnp.float32),
            jax.ShapeDtypeStruct((k, _L), jnp.float32),
        ],
        scratch_shapes=[pltpu.VMEM((1, k), jnp.int32),
                        pltpu.VMEM((1, k), jnp.float32)],
    )(z, cbt, cw)


def _tc_soft_call(z, cbt, n, d, k):
    nblocks = n // ROW_BLOCK
    return pl.pallas_call(
        _tc_soft_body,
        grid=(nblocks,),
        in_specs=[
            pl.BlockSpec((ROW_BLOCK, d), lambda i: (i, 0)),
            pl.BlockSpec((d, k), lambda i: (0, 0)),
        ],
        out_specs=[
            pl.BlockSpec((ROW_BLOCK, k), lambda i: (i, 0)),
        ],
        out_shape=[
            jax.ShapeDtypeStruct((n, k), jnp.float32),
        ],
        scratch_shapes=[pltpu.VMEM((1, k), jnp.float32)],
    )(z, cbt)


_NC, _NS, _L = 2, 16, 16
_NW = _NC * _NS
_CHUNK = 64
_NBUF = 6


def _sc_body(n, k, d, idx_hbm, cb_hbm, z_hbm, ridx_hbm, deadm_hbm,
             quant_hbm, ul_hbm,
             idx_refs, bufs, ridx_v, rl_v, cbl_v, dm_v,
             sem_g, sem_o, sem_r):
    rpw = n // _NW
    nch = rpw // _CHUNK
    kpw = k // _NW
    sid = lax.axis_index("s")
    wid = sid * _NC + lax.axis_index("c")
    base = wid * rpw
    kbase = wid * kpw

    with jax.named_scope("sc_stage"):
        for c in range(nch):
            pltpu.sync_copy(idx_hbm.at[pl.ds(base + c * _CHUNK, _CHUNK)],
                            idx_refs[c])

        pltpu.sync_copy(ridx_hbm.at[pl.ds(kbase, kpw)], ridx_v)
        pltpu.sync_copy(cb_hbm.at[pl.ds(kbase, kpw)], cbl_v)
        pltpu.sync_copy(deadm_hbm.at[pl.ds(kbase, kpw)], dm_v)
        rl_cp = pltpu.async_copy(z_hbm.at[ridx_v], rl_v, sem_r)

    nbuf = len(bufs)
    gcp = [None] * nch
    ocp = [None] * nch
    for c in range(min(nbuf, nch)):
        gcp[c] = pltpu.async_copy(cb_hbm.at[idx_refs[c]], bufs[c], sem_g)

    with jax.named_scope("sc_ul"):
        rl_cp.wait()

        def row_body(r, carry):
            dm16 = dm_v[r, pl.ds(0, _L)]
            for cc in range(d // _L):
                sl = pl.ds(cc * _L, _L)
                rl_v[r, sl] = (rl_v[r, sl] - cbl_v[r, sl]) * dm16
            return carry

        lax.fori_loop(0, kpw, row_body, 0)
        ul_cp = pltpu.async_copy(rl_v, ul_hbm.at[pl.ds(kbase, kpw)], sem_r)

    with jax.named_scope("sc_qgather"):
        waited = [False] * nch
        for c in range(nch):
            gcp[c].wait()
            ocp[c] = pltpu.async_copy(
                bufs[c % nbuf], quant_hbm.at[pl.ds(base + c * _CHUNK, _CHUNK)],
                sem_o)
            nxt = c + nbuf
            if nxt < nch:
                ocp[c].wait()
                waited[c] = True
                gcp[nxt] = pltpu.async_copy(
                    cb_hbm.at[idx_refs[nxt]], bufs[nxt % nbuf], sem_g)
        for c in range(nch):
            if not waited[c]:
                ocp[c].wait()
        ul_cp.wait()


def _sc_call(idx, cb, z, ridx, deadm, n, d, k):
    rpw = n // _NW
    kpw = k // _NW
    mesh = plsc.VectorSubcoreMesh(core_axis_name="c", subcore_axis_name="s")
    return pl.kernel(
        functools.partial(_sc_body, n, k, d),
        out_type=(jax.ShapeDtypeStruct((n, d), jnp.float32),
                  jax.ShapeDtypeStruct((k, d), jnp.float32)),
        mesh=mesh,
        scratch_types=[
            [pltpu.VMEM((_CHUNK,), jnp.int32)
             for _ in range(rpw // _CHUNK)],
            [pltpu.VMEM((_CHUNK, d), jnp.float32) for _ in range(_NBUF)],
            pltpu.VMEM((kpw,), jnp.int32),
            pltpu.VMEM((kpw, d), jnp.float32),
            pltpu.VMEM((kpw, d), jnp.float32),
            pltpu.VMEM((kpw, _L), jnp.float32),
            pltpu.SemaphoreType.DMA,
            pltpu.SemaphoreType.DMA,
            pltpu.SemaphoreType.DMA,
        ],
    )(idx, cb, z, ridx, deadm)


def kernel(continuous_latent, codebook, codebook_weights, rand_idx):
    b, a, d = continuous_latent.shape
    k = codebook.shape[0]
    n = b * a
    z = continuous_latent.reshape(n, d)
    cbt = codebook.T
    cw = codebook_weights.reshape(1, k)

    idx2, nw2, deadm = _tc_idx_call(z, cbt, cw, n, d, k)
    quant, useless = _sc_call(idx2.reshape(n), codebook, z, rand_idx,
                              deadm, n, d, k)
    (soft,) = _tc_soft_call(z, cbt, n, d, k)
    return (quant.reshape(b, a, d), soft.reshape(b, a, k),
            nw2.reshape(k), useless)


# speedup vs baseline: 1.6116x; 1.6116x over previous
import functools

import jax
import jax.numpy as jnp
from jax import lax
from jax.experimental import pallas as pl
from jax.experimental.pallas import tpu as pltpu
from jax.experimental.pallas import tpu_sc as plsc

ROW_BLOCK = 512


def _dist_block(z, cbt, c2):
    xc = lax.dot_general(z, cbt, (((1,), (0,)), ((), ())),
                         preferred_element_type=jnp.float32)
    x2 = jnp.sum(z * z, axis=1, keepdims=True)
    return (x2 + c2) - 2.0 * xc


def _tc_body(nblocks, k, z_ref, cbt_ref, cw_ref, soft_ref, quant_ref,
             nw_ref, deadm_ref, hist_ref, c2_ref):
    i = pl.program_id(0)
    cbt = cbt_ref[...]

    @pl.when(i == 0)
    def _():
        c2_ref[...] = jnp.sum(cbt * cbt, axis=0, keepdims=True)

    dist = _dist_block(z_ref[...], cbt, c2_ref[...])
    rowmin = jnp.min(dist, axis=1, keepdims=True)
    t = -100.0 * dist
    m = -100.0 * rowmin
    e = jnp.exp(t - m)
    soft_ref[...] = e * (1.0 / jnp.sum(e, axis=1, keepdims=True))

    mask = dist == rowmin
    iota = lax.broadcasted_iota(jnp.int32, dist.shape, 1)
    idxv = jnp.min(jnp.where(mask, iota, k), axis=1, keepdims=True)
    onehot = (idxv == iota).astype(jnp.float32)
    quant_ref[...] = lax.dot_general(
        onehot, cbt, (((1,), (1,)), ((), ())),
        precision=None,
        preferred_element_type=jnp.float32)

    cnt = jnp.sum(onehot, axis=0, keepdims=True)

    @pl.when(i == 0)
    def _():
        hist_ref[...] = cnt

    @pl.when(i > 0)
    def _():
        hist_ref[...] += cnt

    @pl.when(i == nblocks - 1)
    def _():
        nw = cw_ref[...] + hist_ref[...]
        nw_ref[...] = nw
        total = jnp.sum(nw)
        deadf = jnp.where(nw < total / (100.0 * k), 1.0, 0.0)
        deadm_ref[...] = jnp.broadcast_to(deadf.reshape(k, 1),
                                          deadm_ref.shape)


def _tc_call(z, cbt, cw, n, d, k):
    nblocks = n // ROW_BLOCK
    return pl.pallas_call(
        functools.partial(_tc_body, nblocks, k),
        grid=(nblocks,),
        in_specs=[
            pl.BlockSpec((ROW_BLOCK, d), lambda i: (i, 0)),
            pl.BlockSpec((d, k), lambda i: (0, 0)),
            pl.BlockSpec((1, k), lambda i: (0, 0)),
        ],
        out_specs=[
            pl.BlockSpec((ROW_BLOCK, k), lambda i: (i, 0)),
            pl.BlockSpec((ROW_BLOCK, d), lambda i: (i, 0)),
            pl.BlockSpec((1, k), lambda i: (0, 0)),
            pl.BlockSpec((k, _L), lambda i: (0, 0)),
        ],
        out_shape=[
            jax.ShapeDtypeStruct((n, k), jnp.float32),
            jax.ShapeDtypeStruct((n, d), jnp.float32),
            jax.ShapeDtypeStruct((1, k), jnp.float32),
            jax.ShapeDtypeStruct((k, _L), jnp.float32),
        ],
        scratch_shapes=[pltpu.VMEM((1, k), jnp.float32),
                        pltpu.VMEM((1, k), jnp.float32)],
    )(z, cbt, cw)


_NC, _NS, _L = 2, 16, 16
_NW = _NC * _NS


def _sc_body(n, k, d, cb_hbm, z_hbm, ridx_hbm, deadm_hbm, ul_hbm,
             ridx_v, rl_v, cbl_v, dm_v, sem_r):
    kpw = k // _NW
    wid = lax.axis_index("s") * _NC + lax.axis_index("c")
    kbase = wid * kpw

    pltpu.sync_copy(ridx_hbm.at[pl.ds(kbase, kpw)], ridx_v)
    rl_cp = pltpu.async_copy(z_hbm.at[ridx_v], rl_v, sem_r)
    pltpu.sync_copy(cb_hbm.at[pl.ds(kbase, kpw)], cbl_v)
    pltpu.sync_copy(deadm_hbm.at[pl.ds(kbase, kpw)], dm_v)
    rl_cp.wait()

    def row_body(r, carry):
        dm16 = dm_v[r, pl.ds(0, _L)]
        for cc in range(d // _L):
            sl = pl.ds(cc * _L, _L)
            rl_v[r, sl] = (rl_v[r, sl] - cbl_v[r, sl]) * dm16
        return carry

    lax.fori_loop(0, kpw, row_body, 0)
    pltpu.sync_copy(rl_v, ul_hbm.at[pl.ds(kbase, kpw)])


def _sc_call(cb, z, ridx, deadm, n, d, k):
    kpw = k // _NW
    mesh = plsc.VectorSubcoreMesh(core_axis_name="c", subcore_axis_name="s")
    return pl.kernel(
        functools.partial(_sc_body, n, k, d),
        out_type=jax.ShapeDtypeStruct((k, d), jnp.float32),
        mesh=mesh,
        scratch_types=[
            pltpu.VMEM((kpw,), jnp.int32),
            pltpu.VMEM((kpw, d), jnp.float32),
            pltpu.VMEM((kpw, d), jnp.float32),
            pltpu.VMEM((kpw, _L), jnp.float32),
            pltpu.SemaphoreType.DMA,
        ],
    )(cb, z, ridx, deadm)


def kernel(continuous_latent, codebook, codebook_weights, rand_idx):
    b, a, d = continuous_latent.shape
    k = codebook.shape[0]
    n = b * a
    z = continuous_latent.reshape(n, d)
    cbt = codebook.T
    cw = codebook_weights.reshape(1, k)

    soft, quant, nw2, deadm = _tc_call(z, cbt, cw, n, d, k)
    useless = _sc_call(codebook, z, rand_idx, deadm, n, d, k)
    return (quant.reshape(b, a, d), soft.reshape(b, a, k),
            nw2.reshape(k), useless)


# speedup vs baseline: 1.7469x; 1.0839x over previous
import functools

import jax
import jax.numpy as jnp
from jax import lax
from jax.experimental import pallas as pl
from jax.experimental.pallas import tpu as pltpu
from jax.experimental.pallas import tpu_sc as plsc

ROW_BLOCK = 1024


def _dist_block(z, cbt, c2):
    xc = lax.dot_general(z, cbt, (((1,), (0,)), ((), ())),
                         preferred_element_type=jnp.float32)
    x2 = jnp.sum(z * z, axis=1, keepdims=True)
    return (x2 + c2) - 2.0 * xc


def _tc_body(nblocks, k, z_ref, cbt_ref, cw_ref, soft_ref, quant_ref,
             nw_ref, deadm_ref, hist_ref, c2_ref):
    i = pl.program_id(0)
    cbt = cbt_ref[...]

    @pl.when(i == 0)
    def _():
        c2_ref[...] = jnp.sum(cbt * cbt, axis=0, keepdims=True)

    dist = _dist_block(z_ref[...], cbt, c2_ref[...])
    rowmin = jnp.min(dist, axis=1, keepdims=True)
    t = -100.0 * dist
    m = -100.0 * rowmin
    e = jnp.exp(t - m)
    soft_ref[...] = e * (1.0 / jnp.sum(e, axis=1, keepdims=True))

    mask = dist == rowmin
    iota = lax.broadcasted_iota(jnp.int32, dist.shape, 1)
    idxv = jnp.min(jnp.where(mask, iota, k), axis=1, keepdims=True)
    onehot = (idxv == iota).astype(jnp.float32)
    quant_ref[...] = lax.dot_general(
        onehot, cbt, (((1,), (1,)), ((), ())),
        precision=None,
        preferred_element_type=jnp.float32)

    cnt = jnp.sum(onehot, axis=0, keepdims=True)

    @pl.when(i == 0)
    def _():
        hist_ref[...] = cnt

    @pl.when(i > 0)
    def _():
        hist_ref[...] += cnt

    @pl.when(i == nblocks - 1)
    def _():
        nw = cw_ref[...] + hist_ref[...]
        nw_ref[...] = nw
        total = jnp.sum(nw)
        deadf = jnp.where(nw < total / (100.0 * k), 1.0, 0.0)
        deadm_ref[...] = jnp.broadcast_to(deadf.reshape(k, 1),
                                          deadm_ref.shape)


def _tc_call(z, cbt, cw, n, d, k):
    nblocks = n // ROW_BLOCK
    return pl.pallas_call(
        functools.partial(_tc_body, nblocks, k),
        grid=(nblocks,),
        in_specs=[
            pl.BlockSpec((ROW_BLOCK, d), lambda i: (i, 0)),
            pl.BlockSpec((d, k), lambda i: (0, 0)),
            pl.BlockSpec((1, k), lambda i: (0, 0)),
        ],
        out_specs=[
            pl.BlockSpec((ROW_BLOCK, k), lambda i: (i, 0)),
            pl.BlockSpec((ROW_BLOCK, d), lambda i: (i, 0)),
            pl.BlockSpec((1, k), lambda i: (0, 0)),
            pl.BlockSpec((k, _L), lambda i: (0, 0)),
        ],
        out_shape=[
            jax.ShapeDtypeStruct((n, k), jnp.float32),
            jax.ShapeDtypeStruct((n, d), jnp.float32),
            jax.ShapeDtypeStruct((1, k), jnp.float32),
            jax.ShapeDtypeStruct((k, _L), jnp.float32),
        ],
        scratch_shapes=[pltpu.VMEM((1, k), jnp.float32),
                        pltpu.VMEM((1, k), jnp.float32)],
    )(z, cbt, cw)


_NC, _NS, _L = 2, 16, 16
_NW = _NC * _NS


def _sc_body(n, k, d, cb_hbm, z_hbm, ridx_hbm, deadm_hbm, ul_hbm,
             ridx_v, rl_v, cbl_v, dm_v, sem_r):
    kpw = k // _NW
    wid = lax.axis_index("s") * _NC + lax.axis_index("c")
    kbase = wid * kpw

    pltpu.sync_copy(ridx_hbm.at[pl.ds(kbase, kpw)], ridx_v)
    rl_cp = pltpu.async_copy(z_hbm.at[ridx_v], rl_v, sem_r)
    pltpu.sync_copy(cb_hbm.at[pl.ds(kbase, kpw)], cbl_v)
    pltpu.sync_copy(deadm_hbm.at[pl.ds(kbase, kpw)], dm_v)
    rl_cp.wait()

    def row_body(r, carry):
        dm16 = dm_v[r, pl.ds(0, _L)]
        for cc in range(d // _L):
            sl = pl.ds(cc * _L, _L)
            rl_v[r, sl] = (rl_v[r, sl] - cbl_v[r, sl]) * dm16
        return carry

    lax.fori_loop(0, kpw, row_body, 0)
    pltpu.sync_copy(rl_v, ul_hbm.at[pl.ds(kbase, kpw)])


def _sc_call(cb, z, ridx, deadm, n, d, k):
    kpw = k // _NW
    mesh = plsc.VectorSubcoreMesh(core_axis_name="c", subcore_axis_name="s")
    return pl.kernel(
        functools.partial(_sc_body, n, k, d),
        out_type=jax.ShapeDtypeStruct((k, d), jnp.float32),
        mesh=mesh,
        scratch_types=[
            pltpu.VMEM((kpw,), jnp.int32),
            pltpu.VMEM((kpw, d), jnp.float32),
            pltpu.VMEM((kpw, d), jnp.float32),
            pltpu.VMEM((kpw, _L), jnp.float32),
            pltpu.SemaphoreType.DMA,
        ],
    )(cb, z, ridx, deadm)


def kernel(continuous_latent, codebook, codebook_weights, rand_idx):
    b, a, d = continuous_latent.shape
    k = codebook.shape[0]
    n = b * a
    z = continuous_latent.reshape(n, d)
    cbt = codebook.T
    cw = codebook_weights.reshape(1, k)

    soft, quant, nw2, deadm = _tc_call(z, cbt, cw, n, d, k)
    useless = _sc_call(codebook, z, rand_idx, deadm, n, d, k)
    return (quant.reshape(b, a, d), soft.reshape(b, a, k),
            nw2.reshape(k), useless)


# speedup vs baseline: 1.7863x; 1.0226x over previous
import functools

import jax
import jax.numpy as jnp
from jax import lax
from jax.experimental import pallas as pl
from jax.experimental.pallas import tpu as pltpu
from jax.experimental.pallas import tpu_sc as plsc

ROW_BLOCK = 2048


def _dist_block(z, cbt, c2):
    xc = lax.dot_general(z, cbt, (((1,), (0,)), ((), ())),
                         preferred_element_type=jnp.float32)
    x2 = jnp.sum(z * z, axis=1, keepdims=True)
    return (x2 + c2) - 2.0 * xc


def _tc_body(nblocks, k, z_ref, cbt_ref, cw_ref, soft_ref, quant_ref,
             nw_ref, deadm_ref, hist_ref, c2_ref):
    i = pl.program_id(0)
    cbt = cbt_ref[...]

    @pl.when(i == 0)
    def _():
        c2_ref[...] = jnp.sum(cbt * cbt, axis=0, keepdims=True)

    dist = _dist_block(z_ref[...], cbt, c2_ref[...])
    rowmin = jnp.min(dist, axis=1, keepdims=True)
    t = -100.0 * dist
    m = -100.0 * rowmin
    e = jnp.exp(t - m)
    soft_ref[...] = e * (1.0 / jnp.sum(e, axis=1, keepdims=True))

    mask = dist == rowmin
    iota = lax.broadcasted_iota(jnp.int32, dist.shape, 1)
    idxv = jnp.min(jnp.where(mask, iota, k), axis=1, keepdims=True)
    onehot = (idxv == iota).astype(jnp.float32)
    quant_ref[...] = lax.dot_general(
        onehot, cbt, (((1,), (1,)), ((), ())),
        precision=None,
        preferred_element_type=jnp.float32)

    cnt = jnp.sum(onehot, axis=0, keepdims=True)

    @pl.when(i == 0)
    def _():
        hist_ref[...] = cnt

    @pl.when(i > 0)
    def _():
        hist_ref[...] += cnt

    @pl.when(i == nblocks - 1)
    def _():
        nw = cw_ref[...] + hist_ref[...]
        nw_ref[...] = nw
        total = jnp.sum(nw)
        deadf = jnp.where(nw < total / (100.0 * k), 1.0, 0.0)
        deadm_ref[...] = jnp.broadcast_to(deadf.reshape(k, 1),
                                          deadm_ref.shape)


def _tc_call(z, cbt, cw, n, d, k):
    nblocks = n // ROW_BLOCK
    return pl.pallas_call(
        functools.partial(_tc_body, nblocks, k),
        grid=(nblocks,),
        in_specs=[
            pl.BlockSpec((ROW_BLOCK, d), lambda i: (i, 0)),
            pl.BlockSpec((d, k), lambda i: (0, 0)),
            pl.BlockSpec((1, k), lambda i: (0, 0)),
        ],
        out_specs=[
            pl.BlockSpec((ROW_BLOCK, k), lambda i: (i, 0)),
            pl.BlockSpec((ROW_BLOCK, d), lambda i: (i, 0)),
            pl.BlockSpec((1, k), lambda i: (0, 0)),
            pl.BlockSpec((k, _L), lambda i: (0, 0)),
        ],
        out_shape=[
            jax.ShapeDtypeStruct((n, k), jnp.float32),
            jax.ShapeDtypeStruct((n, d), jnp.float32),
            jax.ShapeDtypeStruct((1, k), jnp.float32),
            jax.ShapeDtypeStruct((k, _L), jnp.float32),
        ],
        scratch_shapes=[pltpu.VMEM((1, k), jnp.float32),
                        pltpu.VMEM((1, k), jnp.float32)],
    )(z, cbt, cw)


_NC, _NS, _L = 2, 16, 16
_NW = _NC * _NS


def _sc_body(n, k, d, cb_hbm, z_hbm, ridx_hbm, deadm_hbm, ul_hbm,
             ridx_v, rl_v, cbl_v, dm_v, sem_r):
    kpw = k // _NW
    wid = lax.axis_index("s") * _NC + lax.axis_index("c")
    kbase = wid * kpw

    pltpu.sync_copy(ridx_hbm.at[pl.ds(kbase, kpw)], ridx_v)
    rl_cp = pltpu.async_copy(z_hbm.at[ridx_v], rl_v, sem_r)
    pltpu.sync_copy(cb_hbm.at[pl.ds(kbase, kpw)], cbl_v)
    pltpu.sync_copy(deadm_hbm.at[pl.ds(kbase, kpw)], dm_v)
    rl_cp.wait()

    def row_body(r, carry):
        dm16 = dm_v[r, pl.ds(0, _L)]
        for cc in range(d // _L):
            sl = pl.ds(cc * _L, _L)
            rl_v[r, sl] = (rl_v[r, sl] - cbl_v[r, sl]) * dm16
        return carry

    lax.fori_loop(0, kpw, row_body, 0)
    pltpu.sync_copy(rl_v, ul_hbm.at[pl.ds(kbase, kpw)])


def _sc_call(cb, z, ridx, deadm, n, d, k):
    kpw = k // _NW
    mesh = plsc.VectorSubcoreMesh(core_axis_name="c", subcore_axis_name="s")
    return pl.kernel(
        functools.partial(_sc_body, n, k, d),
        out_type=jax.ShapeDtypeStruct((k, d), jnp.float32),
        mesh=mesh,
        scratch_types=[
            pltpu.VMEM((kpw,), jnp.int32),
            pltpu.VMEM((kpw, d), jnp.float32),
            pltpu.VMEM((kpw, d), jnp.float32),
            pltpu.VMEM((kpw, _L), jnp.float32),
            pltpu.SemaphoreType.DMA,
        ],
    )(cb, z, ridx, deadm)


def kernel(continuous_latent, codebook, codebook_weights, rand_idx):
    b, a, d = continuous_latent.shape
    k = codebook.shape[0]
    n = b * a
    z = continuous_latent.reshape(n, d)
    cbt = codebook.T
    cw = codebook_weights.reshape(1, k)

    soft, quant, nw2, deadm = _tc_call(z, cbt, cw, n, d, k)
    useless = _sc_call(codebook, z, rand_idx, deadm, n, d, k)
    return (quant.reshape(b, a, d), soft.reshape(b, a, k),
            nw2.reshape(k), useless)
